# Initial kernel scaffold; baseline (speedup 1.0000x reference)
import functools

import jax
import jax.numpy as jnp
from jax import lax
from jax.experimental import pallas as pl
from jax.experimental.pallas import tpu as pltpu
from jax.experimental.pallas import tpu_sc as plsc

_LANES = 16
_BLK = 64


def _grid_prep_body(raw_ref, gl_ref, gr_ref, w_ref, *, n_batch, f_bins):
    x = raw_ref[...]
    inc = jax.nn.softplus(x)
    i_idx = lax.broadcasted_iota(jnp.int32, (f_bins, f_bins), 0)
    j_idx = lax.broadcasted_iota(jnp.int32, (f_bins, f_bins), 1)
    tri = (i_idx < j_idx).astype(jnp.float32)
    cum = jnp.dot(inc, tri, preferred_element_type=jnp.float32)
    col = lax.broadcasted_iota(jnp.int32, (1, f_bins), 1)
    total = jnp.sum(jnp.where(col == f_bins - 1, cum, 0.0))
    total = jnp.maximum(total, 1e-6)
    grid = jnp.clip(cum / total, 0.0, 1.0)
    scaled = jnp.clip(grid * (f_bins - 1), 0.0, f_bins - 1.0)
    left_f = jnp.floor(scaled)
    left = left_f.astype(jnp.int32)
    right = jnp.minimum(left + 1, f_bins - 1)
    w_ref[...] = scaled - left_f
    row_off = f_bins * lax.broadcasted_iota(jnp.int32, (n_batch, f_bins), 0)
    gl_ref[...] = left + row_off
    gr_ref[...] = right + row_off


def _make_sc_warp(n_batch, f_bins, d_model):
    info = plsc.get_sparse_core_info()
    nc, ns = info.num_cores, info.num_subcores
    assert n_batch == nc * ns, (n_batch, nc, ns)
    assert f_bins % _BLK == 0 and d_model % _LANES == 0
    mesh = plsc.VectorSubcoreMesh(core_axis_name="c", subcore_axis_name="s")

    @functools.partial(
        pl.kernel,
        mesh=mesh,
        out_type=jax.ShapeDtypeStruct((n_batch * f_bins, d_model), jnp.float32),
        scratch_types=[
            pltpu.VMEM((f_bins,), jnp.int32),
            pltpu.VMEM((f_bins,), jnp.int32),
            pltpu.VMEM((f_bins,), jnp.float32),
            pltpu.VMEM((_BLK, d_model), jnp.float32),
            pltpu.VMEM((_BLK, d_model), jnp.float32),
            pltpu.VMEM((_BLK, d_model), jnp.float32),
            pltpu.SemaphoreType.DMA,
            pltpu.SemaphoreType.DMA,
        ],
    )
    def sc_warp(x_hbm, gl_hbm, gr_hbm, w_hbm, out_hbm,
                lidx, ridx, wv, lbuf, rbuf, obuf, lsem, rsem):
        n = lax.axis_index("s") * nc + lax.axis_index("c")
        pltpu.sync_copy(gl_hbm.at[n], lidx)
        pltpu.sync_copy(gr_hbm.at[n], ridx)
        pltpu.sync_copy(w_hbm, wv)
        base = n * f_bins

        def blk_body(b, carry):
            off = b * _BLK
            cl = pltpu.async_copy(x_hbm.at[lidx.at[pl.ds(off, _BLK)]], lbuf, lsem)
            cr = pltpu.async_copy(x_hbm.at[ridx.at[pl.ds(off, _BLK)]], rbuf, rsem)
            cl.wait()
            cr.wait()

            def row_body(i, rcarry):
                wsplat = plsc.load_gather(
                    wv, [jnp.full((_LANES,), off + i, jnp.int32)])
                for j in range(d_model // _LANES):
                    sl = pl.ds(j * _LANES, _LANES)
                    l = lbuf[i, sl]
                    r = rbuf[i, sl]
                    obuf[i, sl] = l + wsplat * (r - l)
                return rcarry

            lax.fori_loop(0, _BLK, row_body, 0)
            pltpu.sync_copy(obuf, out_hbm.at[pl.ds(base + off, _BLK)])
            return carry

        lax.fori_loop(0, f_bins // _BLK, blk_body, 0)

    return sc_warp


def kernel(freq_features, raw_increments):
    orig_ndim = freq_features.ndim
    if orig_ndim == 2:
        freq = freq_features[:, :, None]
    else:
        freq = freq_features
    n_batch, f_bins, d_model = freq.shape

    raw_pad = jnp.concatenate(
        [raw_increments, jnp.zeros((1,), raw_increments.dtype)]
    ).reshape(1, f_bins)
    gl, gr, w = pl.pallas_call(
        functools.partial(_grid_prep_body, n_batch=n_batch, f_bins=f_bins),
        out_shape=[
            jax.ShapeDtypeStruct((n_batch, f_bins), jnp.int32),
            jax.ShapeDtypeStruct((n_batch, f_bins), jnp.int32),
            jax.ShapeDtypeStruct((1, f_bins), jnp.float32),
        ],
    )(raw_pad)

    x_flat = freq.reshape(n_batch * f_bins, d_model)
    sc_warp = _make_sc_warp(n_batch, f_bins, d_model)
    out_flat = sc_warp(x_flat, gl, gr, w.reshape(f_bins))
    out = out_flat.reshape(n_batch, f_bins, d_model)
    if orig_ndim == 2:
        return out[:, :, 0]
    return out


# speedup vs baseline: 1.6470x; 1.6470x over previous
import functools

import jax
import jax.numpy as jnp
from jax import lax
from jax.experimental import pallas as pl
from jax.experimental.pallas import tpu as pltpu
from jax.experimental.pallas import tpu_sc as plsc

_LANES = 16
_BLK = 64


def _grid_prep_body(raw_ref, gl_ref, gr_ref, w_ref, *, n_batch, f_bins):
    x = raw_ref[...]
    inc = jax.nn.softplus(x)
    i_idx = lax.broadcasted_iota(jnp.int32, (f_bins, f_bins), 0)
    j_idx = lax.broadcasted_iota(jnp.int32, (f_bins, f_bins), 1)
    tri = (i_idx < j_idx).astype(jnp.float32)
    cum = jnp.dot(inc, tri, preferred_element_type=jnp.float32)
    col = lax.broadcasted_iota(jnp.int32, (1, f_bins), 1)
    total = jnp.sum(jnp.where(col == f_bins - 1, cum, 0.0))
    total = jnp.maximum(total, 1e-6)
    grid = jnp.clip(cum / total, 0.0, 1.0)
    scaled = jnp.clip(grid * (f_bins - 1), 0.0, f_bins - 1.0)
    left_f = jnp.floor(scaled)
    left = left_f.astype(jnp.int32)
    right = jnp.minimum(left + 1, f_bins - 1)
    w_row = scaled - left_f
    w_ref[...] = jnp.broadcast_to(jnp.transpose(w_row), (f_bins, _LANES))
    row_off = f_bins * lax.broadcasted_iota(jnp.int32, (n_batch, f_bins), 0)
    gl_ref[...] = left + row_off
    gr_ref[...] = right + row_off


def _make_sc_warp(n_batch, f_bins, d_model):
    info = plsc.get_sparse_core_info()
    nc, ns = info.num_cores, info.num_subcores
    assert n_batch == nc * ns, (n_batch, nc, ns)
    assert f_bins % _BLK == 0 and d_model % _LANES == 0
    mesh = plsc.VectorSubcoreMesh(core_axis_name="c", subcore_axis_name="s")

    @functools.partial(
        pl.kernel,
        mesh=mesh,
        out_type=jax.ShapeDtypeStruct((n_batch * f_bins, d_model), jnp.float32),
        scratch_types=[
            pltpu.VMEM((f_bins,), jnp.int32),
            pltpu.VMEM((f_bins,), jnp.int32),
            pltpu.VMEM((_BLK, _LANES), jnp.float32),
            pltpu.VMEM((_BLK, d_model), jnp.float32),
            pltpu.VMEM((_BLK, d_model), jnp.float32),
            pltpu.VMEM((_BLK, d_model), jnp.float32),
            pltpu.SemaphoreType.DMA,
            pltpu.SemaphoreType.DMA,
            pltpu.SemaphoreType.DMA,
        ],
    )
    def sc_warp(x_hbm, gl_hbm, gr_hbm, w_hbm, out_hbm,
                lidx, ridx, wbuf, lbuf, rbuf, obuf, lsem, rsem, wsem):
        n = lax.axis_index("s") * nc + lax.axis_index("c")
        pltpu.sync_copy(gl_hbm.at[n], lidx)
        pltpu.sync_copy(gr_hbm.at[n], ridx)
        base = n * f_bins

        def blk_body(b, carry):
            off = b * _BLK
            cw = pltpu.async_copy(w_hbm.at[pl.ds(off, _BLK)], wbuf, wsem)
            cl = pltpu.async_copy(x_hbm.at[lidx.at[pl.ds(off, _BLK)]], lbuf, lsem)
            cr = pltpu.async_copy(x_hbm.at[ridx.at[pl.ds(off, _BLK)]], rbuf, rsem)
            cw.wait()
            cl.wait()
            cr.wait()

            def row_body(i, rcarry):
                wsplat = wbuf[i]
                for j in range(d_model // _LANES):
                    sl = pl.ds(j * _LANES, _LANES)
                    l = lbuf[i, sl]
                    r = rbuf[i, sl]
                    obuf[i, sl] = l + wsplat * (r - l)
                return rcarry

            lax.fori_loop(0, _BLK, row_body, 0)
            pltpu.sync_copy(obuf, out_hbm.at[pl.ds(base + off, _BLK)])
            return carry

        lax.fori_loop(0, f_bins // _BLK, blk_body, 0)

    return sc_warp


def kernel(freq_features, raw_increments):
    orig_ndim = freq_features.ndim
    if orig_ndim == 2:
        freq = freq_features[:, :, None]
    else:
        freq = freq_features
    n_batch, f_bins, d_model = freq.shape

    raw_pad = jnp.concatenate(
        [raw_increments, jnp.zeros((1,), raw_increments.dtype)]
    ).reshape(1, f_bins)
    gl, gr, w = pl.pallas_call(
        functools.partial(_grid_prep_body, n_batch=n_batch, f_bins=f_bins),
        out_shape=[
            jax.ShapeDtypeStruct((n_batch, f_bins), jnp.int32),
            jax.ShapeDtypeStruct((n_batch, f_bins), jnp.int32),
            jax.ShapeDtypeStruct((f_bins, _LANES), jnp.float32),
        ],
    )(raw_pad)

    x_flat = freq.reshape(n_batch * f_bins, d_model)
    sc_warp = _make_sc_warp(n_batch, f_bins, d_model)
    out_flat = sc_warp(x_flat, gl, gr, w)
    out = out_flat.reshape(n_batch, f_bins, d_model)
    if orig_ndim == 2:
        return out[:, :, 0]
    return out


# speedup vs baseline: 2.4623x; 1.4950x over previous
import functools

import jax
import jax.numpy as jnp
from jax import lax
from jax.experimental import pallas as pl
from jax.experimental.pallas import tpu as pltpu
from jax.experimental.pallas import tpu_sc as plsc

_LANES = 16
_BLK = 32


def _grid_prep_body(raw_ref, gl_ref, gr_ref, w_ref, *, n_batch, f_bins):
    x = raw_ref[...]
    inc = jax.nn.softplus(x)
    i_idx = lax.broadcasted_iota(jnp.int32, (f_bins, f_bins), 0)
    j_idx = lax.broadcasted_iota(jnp.int32, (f_bins, f_bins), 1)
    tri = (i_idx < j_idx).astype(jnp.float32)
    cum = jnp.dot(inc, tri, preferred_element_type=jnp.float32)
    col = lax.broadcasted_iota(jnp.int32, (1, f_bins), 1)
    total = jnp.sum(jnp.where(col == f_bins - 1, cum, 0.0))
    total = jnp.maximum(total, 1e-6)
    grid = jnp.clip(cum / total, 0.0, 1.0)
    scaled = jnp.clip(grid * (f_bins - 1), 0.0, f_bins - 1.0)
    left_f = jnp.floor(scaled)
    left = left_f.astype(jnp.int32)
    right = jnp.minimum(left + 1, f_bins - 1)
    w_row = scaled - left_f
    w_ref[...] = jnp.broadcast_to(jnp.transpose(w_row), (f_bins, _LANES))
    row_off = f_bins * lax.broadcasted_iota(jnp.int32, (n_batch, f_bins), 0)
    gl_ref[...] = left + row_off
    gr_ref[...] = right + row_off


def _make_sc_warp(n_batch, f_bins, d_model):
    info = plsc.get_sparse_core_info()
    nc, ns = info.num_cores, info.num_subcores
    assert n_batch == nc * ns, (n_batch, nc, ns)
    assert f_bins % _BLK == 0 and d_model % _LANES == 0
    mesh = plsc.VectorSubcoreMesh(core_axis_name="c", subcore_axis_name="s")

    num_blocks = f_bins // _BLK
    assert num_blocks % 2 == 0

    @functools.partial(
        pl.kernel,
        mesh=mesh,
        out_type=jax.ShapeDtypeStruct((n_batch * f_bins, d_model), jnp.float32),
        scratch_types=[
            pltpu.VMEM((f_bins,), jnp.int32),
            pltpu.VMEM((f_bins,), jnp.int32),
            pltpu.VMEM((_BLK, _LANES), jnp.float32),
            pltpu.VMEM((_BLK, _LANES), jnp.float32),
            pltpu.VMEM((_BLK, d_model), jnp.float32),
            pltpu.VMEM((_BLK, d_model), jnp.float32),
            pltpu.VMEM((_BLK, d_model), jnp.float32),
            pltpu.VMEM((_BLK, d_model), jnp.float32),
            pltpu.VMEM((_BLK, d_model), jnp.float32),
            pltpu.SemaphoreType.DMA,
            pltpu.SemaphoreType.DMA,
        ],
    )
    def sc_warp(x_hbm, gl_hbm, gr_hbm, w_hbm, out_hbm,
                lidx, ridx, wbuf0, wbuf1, lbuf0, lbuf1, rbuf0, rbuf1, obuf,
                gsem0, gsem1):
        n = lax.axis_index("s") * nc + lax.axis_index("c")
        pltpu.sync_copy(gl_hbm.at[n], lidx)
        pltpu.sync_copy(gr_hbm.at[n], ridx)
        base = n * f_bins
        slots = ((wbuf0, lbuf0, rbuf0, gsem0), (wbuf1, lbuf1, rbuf1, gsem1))

        def gather_copies(b, wb, lb, rb, sem):
            off = b * _BLK
            return (
                pltpu.make_async_copy(w_hbm.at[pl.ds(off, _BLK)], wb, sem),
                pltpu.make_async_copy(x_hbm.at[lidx.at[pl.ds(off, _BLK)]],
                                      lb, sem),
                pltpu.make_async_copy(x_hbm.at[ridx.at[pl.ds(off, _BLK)]],
                                      rb, sem),
            )

        def issue(b, wb, lb, rb, sem):
            for c in gather_copies(b, wb, lb, rb, sem):
                c.start()

        issue(0, *slots[0])
        issue(1, *slots[1])

        def outer(t, carry):
            for s, (wb, lb, rb, sem) in enumerate(slots):
                b = 2 * t + s
                off = b * _BLK
                for c in gather_copies(b, wb, lb, rb, sem):
                    c.wait()

                def row_body(i, rcarry):
                    wsplat = wb[i]
                    for j in range(d_model // _LANES):
                        sl = pl.ds(j * _LANES, _LANES)
                        l = lb[i, sl]
                        r = rb[i, sl]
                        obuf[i, sl] = l + wsplat * (r - l)
                    return rcarry

                lax.fori_loop(0, _BLK, row_body, 0)

                @pl.when(t < num_blocks // 2 - 1)
                def _():
                    issue(b + 2, wb, lb, rb, sem)

                pltpu.sync_copy(obuf, out_hbm.at[pl.ds(base + off, _BLK)])
            return carry

        lax.fori_loop(0, num_blocks // 2, outer, 0)

    return sc_warp


def kernel(freq_features, raw_increments):
    orig_ndim = freq_features.ndim
    if orig_ndim == 2:
        freq = freq_features[:, :, None]
    else:
        freq = freq_features
    n_batch, f_bins, d_model = freq.shape

    raw_pad = jnp.concatenate(
        [raw_increments, jnp.zeros((1,), raw_increments.dtype)]
    ).reshape(1, f_bins)
    gl, gr, w = pl.pallas_call(
        functools.partial(_grid_prep_body, n_batch=n_batch, f_bins=f_bins),
        out_shape=[
            jax.ShapeDtypeStruct((n_batch, f_bins), jnp.int32),
            jax.ShapeDtypeStruct((n_batch, f_bins), jnp.int32),
            jax.ShapeDtypeStruct((f_bins, _LANES), jnp.float32),
        ],
    )(raw_pad)

    x_flat = freq.reshape(n_batch * f_bins, d_model)
    sc_warp = _make_sc_warp(n_batch, f_bins, d_model)
    out_flat = sc_warp(x_flat, gl, gr, w)
    out = out_flat.reshape(n_batch, f_bins, d_model)
    if orig_ndim == 2:
        return out[:, :, 0]
    return out
